# baseline (speedup 1.0000x reference)
import functools

import jax
import jax.numpy as jnp
from jax import lax
from jax.experimental import pallas as pl
from jax.experimental.pallas import tpu as pltpu
from jax.experimental.pallas import tpu_sc as plsc

N = 10000
NP = 10112
E = 320000
EFULL = E + N
K = 128
NB = 162
EPT = K * NB
EPAD = EPT * 16
H = 8
HL = 4
C = 16
HC = H * C
FL = HL * C
OUT = 128
RPT = NP // 16

_f32 = jnp.float32
_i32 = jnp.int32

_MESH = plsc.VectorSubcoreMesh(
    core_axis_name="c", subcore_axis_name="s", num_cores=2, num_subcores=16)
_SC_PARAMS = pltpu.CompilerParams(
    needs_layout_passes=False, use_tc_tiling_on_sc=False)


def _zero_rows(buf, ncols):
    def body(r, carry):
        for j in range(ncols // 16):
            buf[r, pl.ds(16 * j, 16)] = jnp.zeros((16,), _f32)
        return carry
    lax.fori_loop(0, K, body, 0)


def _zero_shared(src_buf, shared, base):
    for kk in range((RPT + K - 1) // K):
        nrows = min(K, RPT - kk * K)
        pltpu.sync_copy(src_buf.at[pl.ds(0, nrows)],
                        shared.at[pl.ds(base + kk * K, nrows)])


def _copy_out(shared, out_hbm, cid, base):
    for kk in range((RPT + K - 1) // K):
        nrows = min(K, RPT - kk * K)
        pltpu.sync_copy(shared.at[pl.ds(base + kk * K, nrows)],
                        out_hbm.at[cid, pl.ds(base + kk * K, nrows)])


def _edge_pass(xlh_hbm, xrh_hbm, att_hbm, src_hbm, dst_hbm, s_out, d_out,
               att_v, src_idx, dst_idx, xl_rows, xr_rows, msg_buf, dbuf,
               s_sh, d_sh, sem_a, sem_b):
    cid = lax.axis_index("c")
    sid = lax.axis_index("s")
    pltpu.sync_copy(att_hbm.at[pl.ds(cid * FL, FL)], att_v)

    _zero_rows(xl_rows, FL)
    iot0 = lax.iota(_i32, 16)
    zrow = iot0 // 8
    zcol = iot0 % 8
    z16 = jnp.zeros((16,), _f32)

    def zdb(r, carry):
        plsc.store_scatter(dbuf, [zrow + 2 * r, zcol], z16)
        return carry
    lax.fori_loop(0, K // 2, zdb, 0)
    base = sid * RPT
    _zero_shared(xl_rows, s_sh, base)
    _zero_shared(dbuf, d_sh, base)
    plsc.subcore_barrier()

    ebase = sid * EPT
    iot = lax.iota(_i32, 16)
    ones16 = jnp.ones((16,), _f32)
    xl_t = xlh_hbm.at[cid]
    xr_t = xrh_hbm.at[cid]
    att_s = []
    for h in range(HL):
        av = att_v[pl.ds(h * C, C)]
        att_s.append([av[c] for c in range(C)])

    def blk(b, carry):
        off = ebase + b * K
        pltpu.sync_copy(src_hbm.at[pl.ds(off, K)], src_idx)
        pltpu.sync_copy(dst_hbm.at[pl.ds(off, K)], dst_idx)
        cp1 = pltpu.async_copy(xl_t.at[src_idx], xl_rows, sem_a)
        cp2 = pltpu.async_copy(xr_t.at[dst_idx], xr_rows, sem_b)
        cp1.wait()
        cp2.wait()
        def grp(g, gcarry):
            rows = iot + g * 16
            ws = []
            for h in range(HL):
                acc = jnp.zeros((16,), _f32)
                for c in range(C):
                    f = h * C + c
                    cols = jnp.full((16,), f, _i32)
                    xlv = plsc.load_gather(xl_rows, [rows, cols])
                    xrv = plsc.load_gather(xr_rows, [rows, cols])
                    s = xlv + xrv
                    lk = jnp.maximum(s, s * _f32(0.2))
                    acc = acc + lk * att_s[h][c]
                w = jnp.exp(acc)
                ws.append(w)
                plsc.store_scatter(dbuf, [rows, jnp.full((16,), h, _i32)], w)
            plsc.store_scatter(dbuf, [rows, jnp.full((16,), HL, _i32)], ones16)
            for h in range(HL):
                for c in range(C):
                    f = h * C + c
                    cols = jnp.full((16,), f, _i32)
                    v = plsc.load_gather(xl_rows, [rows, cols])
                    plsc.store_scatter(msg_buf, [rows, cols], v * ws[h])
            return gcarry

        lax.fori_loop(0, 8, grp, 0)
        pltpu.sync_copy(msg_buf, s_sh.at[dst_idx], add=True)
        pltpu.sync_copy(dbuf, d_sh.at[dst_idx], add=True)
        return carry

    lax.fori_loop(0, NB, blk, 0)
    plsc.subcore_barrier()
    _copy_out(s_sh, s_out, cid, base)
    _copy_out(d_sh, d_out, cid, base)


_edge_call = functools.partial(
    pl.kernel,
    out_type=[jax.ShapeDtypeStruct((2, NP, FL), _f32),
              jax.ShapeDtypeStruct((2, NP, 8), _f32)],
    mesh=_MESH,
    compiler_params=_SC_PARAMS,
    scratch_types=[
        pltpu.VMEM((FL,), _f32),
        pltpu.VMEM((K,), _i32),
        pltpu.VMEM((K,), _i32),
        pltpu.VMEM((K, FL), _f32),
        pltpu.VMEM((K, FL), _f32),
        pltpu.VMEM((K, FL), _f32),
        pltpu.VMEM((K, 8), _f32),
        pltpu.VMEM_SHARED((NP, FL), _f32),
        pltpu.VMEM_SHARED((NP, 8), _f32),
        pltpu.SemaphoreType.DMA,
        pltpu.SemaphoreType.DMA,
    ],
)(_edge_pass)


def _gcn_pass(xgn_hbm, src_hbm, dst_hbm, o_out,
              src_idx, dst_idx, rows_v, o_sh, sem_a):
    cid = lax.axis_index("c")
    sid = lax.axis_index("s")
    _zero_rows(rows_v, FL)
    base = sid * RPT
    _zero_shared(rows_v, o_sh, base)
    plsc.subcore_barrier()
    ebase = sid * EPT
    xg_t = xgn_hbm.at[cid]

    def blk(b, carry):
        off = ebase + b * K
        pltpu.sync_copy(src_hbm.at[pl.ds(off, K)], src_idx)
        pltpu.sync_copy(dst_hbm.at[pl.ds(off, K)], dst_idx)
        pltpu.async_copy(xg_t.at[src_idx], rows_v, sem_a).wait()
        pltpu.sync_copy(rows_v, o_sh.at[dst_idx], add=True)
        return carry

    lax.fori_loop(0, NB, blk, 0)
    plsc.subcore_barrier()
    _copy_out(o_sh, o_out, cid, base)


_gcn_call = functools.partial(
    pl.kernel,
    out_type=jax.ShapeDtypeStruct((2, NP, FL), _f32),
    mesh=_MESH,
    compiler_params=_SC_PARAMS,
    scratch_types=[
        pltpu.VMEM((K,), _i32),
        pltpu.VMEM((K,), _i32),
        pltpu.VMEM((K, FL), _f32),
        pltpu.VMEM_SHARED((NP, FL), _f32),
        pltpu.SemaphoreType.DMA,
    ],
)(_gcn_pass)


def _mm2_body(x_ref, wl_ref, wr_ref, xl_ref, xr_ref):
    x = x_ref[...]
    xl = jnp.dot(x, wl_ref[...], preferred_element_type=_f32)
    xr = jnp.dot(x, wr_ref[...], preferred_element_type=_f32)
    xl_ref[0] = xl[:, :FL]
    xl_ref[1] = xl[:, FL:]
    xr_ref[0] = xr[:, :FL]
    xr_ref[1] = xr[:, FL:]


def _mid_body(s_ref, den_ref, deg_ref, erep_ref, bias1_ref, gamma_ref,
              beta_ref, pw_ref, wg_ref, xgn_ref, dinv_ref):
    S = s_ref[...]
    denom = den_ref[...]
    deg = deg_ref[...]
    den128 = jnp.dot(denom, erep_ref[...], preferred_element_type=_f32)
    h1 = S / den128 + bias1_ref[...]
    mu = jnp.mean(h1, axis=0, keepdims=True)
    xc = h1 - mu
    var = jnp.mean(xc * xc, axis=0, keepdims=True)
    hbn = xc * lax.rsqrt(var + 1e-5) * gamma_ref[...] + beta_ref[...]
    pw = pw_ref[0, 0]
    hp = jnp.where(hbn >= 0, hbn, pw * hbn)
    xg = jnp.dot(hp, wg_ref[...], preferred_element_type=_f32)
    dinv = lax.rsqrt(jnp.maximum(deg, 1.0))
    xgn = xg * dinv
    xgn_ref[0] = xgn[:, :FL]
    xgn_ref[1] = xgn[:, FL:]
    dinv_ref[...] = dinv


def _fin_body(o_ref, dinv_ref, bg_ref, out_ref):
    dinv = dinv_ref[...]
    bg = bg_ref[...]
    out_ref[:, :FL] = o_ref[0] * dinv + bg[:, :FL]
    out_ref[:, FL:] = o_ref[1] * dinv + bg[:, FL:]


def kernel(x, edge_index, Wl, Wr, att, bias1, gamma, beta, prelu_w, Wg, bias_g):
    xpad = jnp.pad(x, ((0, NP - N), (0, 0)))
    xlh, xrh = pl.pallas_call(
        _mm2_body,
        out_shape=[jax.ShapeDtypeStruct((2, NP, FL), _f32)] * 2,
    )(xpad, Wl, Wr)

    loops = jnp.arange(N, dtype=_i32)
    padi = jnp.full((EPAD - EFULL,), N, _i32)
    srcf = jnp.concatenate([edge_index[0], loops, padi])
    dstf = jnp.concatenate([edge_index[1], loops, padi])

    DEBUG_XLA_EDGE = False
    if DEBUG_XLA_EDGE:
        src0, dst0 = srcf[:EFULL], dstf[:EFULL]
        xlf = jnp.concatenate([xlh[0], xlh[1]], axis=1)[:N].reshape(N, H, C)
        xrf = jnp.concatenate([xrh[0], xrh[1]], axis=1)[:N].reshape(N, H, C)
        e = jax.nn.leaky_relu(xlf[src0] + xrf[dst0], negative_slope=0.2)
        logits = jnp.einsum('ehc,hc->eh', e, att)
        ex = jnp.exp(logits)
        denom = jax.ops.segment_sum(ex, dst0, num_segments=N)
        S = jax.ops.segment_sum(
            xlf[src0].reshape(EFULL, HC) * jnp.repeat(ex, C, axis=1),
            dst0, num_segments=N)
        deg = jax.ops.segment_sum(jnp.ones((EFULL, 1), _f32), dst0,
                                  num_segments=N)
    else:
        s_out, d_out = _edge_call(xlh, xrh, att.reshape(-1), srcf, dstf)
        S = jnp.concatenate([s_out[0, :N], s_out[1, :N]], axis=1)
        denom = jnp.concatenate([d_out[0, :N, 0:HL], d_out[1, :N, 0:HL]],
                                axis=1)
        deg = d_out[0, :N, HL:HL + 1]

    erep = jnp.kron(jnp.eye(H, dtype=_f32), jnp.ones((1, C), _f32))

    xgnh, dinv = pl.pallas_call(
        _mid_body,
        out_shape=[jax.ShapeDtypeStruct((2, N, FL), _f32),
                   jax.ShapeDtypeStruct((N, 1), _f32)],
    )(S, denom, deg, erep, bias1.reshape(1, -1), gamma.reshape(1, -1),
      beta.reshape(1, -1), prelu_w.reshape(1, 1), Wg)

    xgn_pad = jnp.pad(xgnh, ((0, 0), (0, NP - N), (0, 0)))
    o_out = _gcn_call(xgn_pad, srcf, dstf)

    out = pl.pallas_call(
        _fin_body,
        out_shape=jax.ShapeDtypeStruct((N, OUT), _f32),
    )(o_out[:, :N], dinv, bias_g.reshape(1, -1))
    return out


# speedup vs baseline: 2.4565x; 2.4565x over previous
import functools

import jax
import jax.numpy as jnp
from jax import lax
from jax.experimental import pallas as pl
from jax.experimental.pallas import tpu as pltpu
from jax.experimental.pallas import tpu_sc as plsc

N = 10000
NP = 10112
E = 320000
EFULL = E + N
K = 128
NB = 162
EPT = K * NB
EPAD = EPT * 16
H = 8
HL = 4
C = 16
HC = H * C
FL = HL * C
OUT = 128
RPT = NP // 16

_f32 = jnp.float32
_i32 = jnp.int32

_MESH = plsc.VectorSubcoreMesh(
    core_axis_name="c", subcore_axis_name="s", num_cores=2, num_subcores=16)
_SC_PARAMS = pltpu.CompilerParams(
    needs_layout_passes=False, use_tc_tiling_on_sc=False)


def _zero_rows(buf, ncols):
    def body(r, carry):
        for j in range(ncols // 16):
            buf[r, pl.ds(16 * j, 16)] = jnp.zeros((16,), _f32)
        return carry
    lax.fori_loop(0, K, body, 0)


def _zero_shared(src_buf, shared, base):
    for kk in range((RPT + K - 1) // K):
        nrows = min(K, RPT - kk * K)
        pltpu.sync_copy(src_buf.at[pl.ds(0, nrows)],
                        shared.at[pl.ds(base + kk * K, nrows)])


def _copy_out(shared, out_hbm, cid, base):
    for kk in range((RPT + K - 1) // K):
        nrows = min(K, RPT - kk * K)
        pltpu.sync_copy(shared.at[pl.ds(base + kk * K, nrows)],
                        out_hbm.at[cid, pl.ds(base + kk * K, nrows)])


def _edge_pass(xlh_hbm, xrh_hbm, att_hbm, src_hbm, dst_hbm, s_out, d_out,
               att_v, src_idx, dst_idx, xl_rows, xr_rows, msg_buf, dbuf,
               s_sh, d_sh, sem_a, sem_b):
    cid = lax.axis_index("c")
    sid = lax.axis_index("s")
    pltpu.sync_copy(att_hbm.at[pl.ds(cid * FL, FL)], att_v)

    _zero_rows(xl_rows, FL)
    _zero_rows(dbuf, 16)
    base = sid * RPT
    _zero_shared(xl_rows, s_sh, base)
    _zero_shared(dbuf, d_sh, base)
    plsc.subcore_barrier()

    ebase = sid * EPT
    iot = lax.iota(_i32, 16)
    xl_t = xlh_hbm.at[cid]
    xr_t = xrh_hbm.at[cid]
    att_hv = [att_v[pl.ds(h * C, C)] for h in range(HL)]
    lane_is = [iot == h for h in range(HL)]
    deg_col = jnp.where(iot == HL, jnp.ones((16,), _f32),
                        jnp.zeros((16,), _f32))

    def blk(b, carry):
        off = ebase + b * K
        pltpu.sync_copy(src_hbm.at[pl.ds(off, K)], src_idx)
        pltpu.sync_copy(dst_hbm.at[pl.ds(off, K)], dst_idx)
        cp1 = pltpu.async_copy(xl_t.at[src_idx], xl_rows, sem_a)
        cp2 = pltpu.async_copy(xr_t.at[dst_idx], xr_rows, sem_b)
        cp1.wait()
        cp2.wait()
        def edge(e, ecarry):
            wrow = deg_col
            for h in range(HL):
                xlv = xl_rows[e, pl.ds(h * C, C)]
                xrv = xr_rows[e, pl.ds(h * C, C)]
                s = xlv + xrv
                lk = jnp.maximum(s, s * _f32(0.2))
                logit = jnp.sum(lk * att_hv[h])
                wv = jnp.exp(jnp.broadcast_to(logit, (16,)))
                msg_buf[e, pl.ds(h * C, C)] = xlv * wv
                wrow = jnp.where(lane_is[h], wv, wrow)
            dbuf[e, :] = wrow
            return ecarry

        lax.fori_loop(0, K, edge, 0)
        pltpu.sync_copy(msg_buf, s_sh.at[dst_idx], add=True)
        pltpu.sync_copy(dbuf, d_sh.at[dst_idx], add=True)
        return carry

    lax.fori_loop(0, NB, blk, 0)
    plsc.subcore_barrier()
    _copy_out(s_sh, s_out, cid, base)
    _copy_out(d_sh, d_out, cid, base)


_edge_call = functools.partial(
    pl.kernel,
    out_type=[jax.ShapeDtypeStruct((2, NP, FL), _f32),
              jax.ShapeDtypeStruct((2, NP, 16), _f32)],
    mesh=_MESH,
    compiler_params=_SC_PARAMS,
    scratch_types=[
        pltpu.VMEM((FL,), _f32),
        pltpu.VMEM((K,), _i32),
        pltpu.VMEM((K,), _i32),
        pltpu.VMEM((K, FL), _f32),
        pltpu.VMEM((K, FL), _f32),
        pltpu.VMEM((K, FL), _f32),
        pltpu.VMEM((K, 16), _f32),
        pltpu.VMEM_SHARED((NP, FL), _f32),
        pltpu.VMEM_SHARED((NP, 16), _f32),
        pltpu.SemaphoreType.DMA,
        pltpu.SemaphoreType.DMA,
    ],
)(_edge_pass)


def _gcn_pass(xgn_hbm, src_hbm, dst_hbm, o_out,
              src_idx, dst_idx, rows_v, o_sh, sem_a):
    cid = lax.axis_index("c")
    sid = lax.axis_index("s")
    _zero_rows(rows_v, FL)
    base = sid * RPT
    _zero_shared(rows_v, o_sh, base)
    plsc.subcore_barrier()
    ebase = sid * EPT
    xg_t = xgn_hbm.at[cid]

    def blk(b, carry):
        off = ebase + b * K
        pltpu.sync_copy(src_hbm.at[pl.ds(off, K)], src_idx)
        pltpu.sync_copy(dst_hbm.at[pl.ds(off, K)], dst_idx)
        pltpu.async_copy(xg_t.at[src_idx], rows_v, sem_a).wait()
        pltpu.sync_copy(rows_v, o_sh.at[dst_idx], add=True)
        return carry

    lax.fori_loop(0, NB, blk, 0)
    plsc.subcore_barrier()
    _copy_out(o_sh, o_out, cid, base)


_gcn_call = functools.partial(
    pl.kernel,
    out_type=jax.ShapeDtypeStruct((2, NP, FL), _f32),
    mesh=_MESH,
    compiler_params=_SC_PARAMS,
    scratch_types=[
        pltpu.VMEM((K,), _i32),
        pltpu.VMEM((K,), _i32),
        pltpu.VMEM((K, FL), _f32),
        pltpu.VMEM_SHARED((NP, FL), _f32),
        pltpu.SemaphoreType.DMA,
    ],
)(_gcn_pass)


def _mm2_body(x_ref, wl_ref, wr_ref, xl_ref, xr_ref):
    x = x_ref[...]
    xl = jnp.dot(x, wl_ref[...], preferred_element_type=_f32)
    xr = jnp.dot(x, wr_ref[...], preferred_element_type=_f32)
    xl_ref[0] = xl[:, :FL]
    xl_ref[1] = xl[:, FL:]
    xr_ref[0] = xr[:, :FL]
    xr_ref[1] = xr[:, FL:]


def _mid_body(s_ref, den_ref, deg_ref, erep_ref, bias1_ref, gamma_ref,
              beta_ref, pw_ref, wg_ref, xgn_ref, dinv_ref):
    S = s_ref[...]
    denom = den_ref[...]
    deg = deg_ref[...]
    den128 = jnp.dot(denom, erep_ref[...], preferred_element_type=_f32)
    h1 = S / den128 + bias1_ref[...]
    mu = jnp.mean(h1, axis=0, keepdims=True)
    xc = h1 - mu
    var = jnp.mean(xc * xc, axis=0, keepdims=True)
    hbn = xc * lax.rsqrt(var + 1e-5) * gamma_ref[...] + beta_ref[...]
    pw = pw_ref[0, 0]
    hp = jnp.where(hbn >= 0, hbn, pw * hbn)
    xg = jnp.dot(hp, wg_ref[...], preferred_element_type=_f32)
    dinv = lax.rsqrt(jnp.maximum(deg, 1.0))
    xgn = xg * dinv
    xgn_ref[0] = xgn[:, :FL]
    xgn_ref[1] = xgn[:, FL:]
    dinv_ref[...] = dinv


def _fin_body(o_ref, dinv_ref, bg_ref, out_ref):
    dinv = dinv_ref[...]
    bg = bg_ref[...]
    out_ref[:, :FL] = o_ref[0] * dinv + bg[:, :FL]
    out_ref[:, FL:] = o_ref[1] * dinv + bg[:, FL:]


def kernel(x, edge_index, Wl, Wr, att, bias1, gamma, beta, prelu_w, Wg, bias_g):
    xpad = jnp.pad(x, ((0, NP - N), (0, 0)))
    xlh, xrh = pl.pallas_call(
        _mm2_body,
        out_shape=[jax.ShapeDtypeStruct((2, NP, FL), _f32)] * 2,
    )(xpad, Wl, Wr)

    loops = jnp.arange(N, dtype=_i32)
    padi = jnp.full((EPAD - EFULL,), N, _i32)
    srcf = jnp.concatenate([edge_index[0], loops, padi])
    dstf = jnp.concatenate([edge_index[1], loops, padi])

    DEBUG_XLA_EDGE = False
    if DEBUG_XLA_EDGE:
        src0, dst0 = srcf[:EFULL], dstf[:EFULL]
        xlf = jnp.concatenate([xlh[0], xlh[1]], axis=1)[:N].reshape(N, H, C)
        xrf = jnp.concatenate([xrh[0], xrh[1]], axis=1)[:N].reshape(N, H, C)
        e = jax.nn.leaky_relu(xlf[src0] + xrf[dst0], negative_slope=0.2)
        logits = jnp.einsum('ehc,hc->eh', e, att)
        ex = jnp.exp(logits)
        denom = jax.ops.segment_sum(ex, dst0, num_segments=N)
        S = jax.ops.segment_sum(
            xlf[src0].reshape(EFULL, HC) * jnp.repeat(ex, C, axis=1),
            dst0, num_segments=N)
        deg = jax.ops.segment_sum(jnp.ones((EFULL, 1), _f32), dst0,
                                  num_segments=N)
    else:
        s_out, d_out = _edge_call(xlh, xrh, att.reshape(-1), srcf, dstf)
        S = jnp.concatenate([s_out[0, :N], s_out[1, :N]], axis=1)
        denom = jnp.concatenate([d_out[0, :N, 0:HL], d_out[1, :N, 0:HL]],
                                axis=1)
        deg = d_out[0, :N, HL:HL + 1]

    erep = jnp.kron(jnp.eye(H, dtype=_f32), jnp.ones((1, C), _f32))

    xgnh, dinv = pl.pallas_call(
        _mid_body,
        out_shape=[jax.ShapeDtypeStruct((2, N, FL), _f32),
                   jax.ShapeDtypeStruct((N, 1), _f32)],
    )(S, denom, deg, erep, bias1.reshape(1, -1), gamma.reshape(1, -1),
      beta.reshape(1, -1), prelu_w.reshape(1, 1), Wg)

    xgn_pad = jnp.pad(xgnh, ((0, 0), (0, NP - N), (0, 0)))
    o_out = _gcn_call(xgn_pad, srcf, dstf)

    out = pl.pallas_call(
        _fin_body,
        out_shape=jax.ShapeDtypeStruct((N, OUT), _f32),
    )(o_out[:, :N], dinv, bias_g.reshape(1, -1))
    return out


# speedup vs baseline: 2.9509x; 1.2012x over previous
import functools

import jax
import jax.numpy as jnp
from jax import lax
from jax.experimental import pallas as pl
from jax.experimental.pallas import tpu as pltpu
from jax.experimental.pallas import tpu_sc as plsc

N = 10000
NP = 10112
E = 320000
EFULL = E + N
K = 128
NB = 162
EPT = K * NB
EPAD = EPT * 16
H = 8
HL = 4
C = 16
HC = H * C
FL = HL * C
OUT = 128
RPT = NP // 16

_f32 = jnp.float32
_i32 = jnp.int32

_MESH = plsc.VectorSubcoreMesh(
    core_axis_name="c", subcore_axis_name="s", num_cores=2, num_subcores=16)
_SC_PARAMS = pltpu.CompilerParams(
    needs_layout_passes=False, use_tc_tiling_on_sc=False)


def _zero_rows(buf, ncols):
    def body(r, carry):
        for j in range(ncols // 16):
            buf[r, pl.ds(16 * j, 16)] = jnp.zeros((16,), _f32)
        return carry
    lax.fori_loop(0, K, body, 0)


def _zero_shared(src_buf, shared, base):
    for kk in range((RPT + K - 1) // K):
        nrows = min(K, RPT - kk * K)
        pltpu.sync_copy(src_buf.at[pl.ds(0, nrows)],
                        shared.at[pl.ds(base + kk * K, nrows)])


def _copy_out(shared, out_hbm, cid, base):
    for kk in range((RPT + K - 1) // K):
        nrows = min(K, RPT - kk * K)
        pltpu.sync_copy(shared.at[pl.ds(base + kk * K, nrows)],
                        out_hbm.at[cid, pl.ds(base + kk * K, nrows)])


def _edge_pass(xlh_hbm, xrh_hbm, att_hbm, src_hbm, dst_hbm, s_out, d_out,
               att_v, src_idx0, src_idx1, dst_idx0, dst_idx1,
               xl0, xl1, xr0, xr1, msg_buf, dbuf,
               s_sh, d_sh, sem_l0, sem_l1, sem_r0, sem_r1):
    cid = lax.axis_index("c")
    sid = lax.axis_index("s")
    pltpu.sync_copy(att_hbm.at[pl.ds(cid * FL, FL)], att_v)

    _zero_rows(xl0, FL)
    _zero_rows(dbuf, 16)
    base = sid * RPT
    _zero_shared(xl0, s_sh, base)
    _zero_shared(dbuf, d_sh, base)
    plsc.subcore_barrier()

    ebase = sid * EPT
    iot = lax.iota(_i32, 16)
    xl_t = xlh_hbm.at[cid]
    xr_t = xrh_hbm.at[cid]
    srcb = [src_idx0, src_idx1]
    dstb = [dst_idx0, dst_idx1]
    xlb = [xl0, xl1]
    xrb = [xr0, xr1]
    sl = [sem_l0, sem_l1]
    sr = [sem_r0, sem_r1]
    att_hv = [att_v[pl.ds(h * C, C)] for h in range(HL)]
    lane_is = [iot == h for h in range(HL)]
    deg_col = jnp.where(iot == HL, jnp.ones((16,), _f32),
                        jnp.zeros((16,), _f32))

    def load_idx(b, p):
        off = ebase + b * K
        pltpu.sync_copy(src_hbm.at[pl.ds(off, K)], srcb[p])
        pltpu.sync_copy(dst_hbm.at[pl.ds(off, K)], dstb[p])

    def fire(p):
        pltpu.async_copy(xl_t.at[srcb[p]], xlb[p], sl[p])
        pltpu.async_copy(xr_t.at[dstb[p]], xrb[p], sr[p])

    def drain(p):
        pltpu.make_async_copy(xl_t.at[srcb[p]], xlb[p], sl[p]).wait()
        pltpu.make_async_copy(xr_t.at[dstb[p]], xrb[p], sr[p]).wait()

    load_idx(0, 0)
    fire(0)

    def pair(i, carry):
        for p in (0, 1):
            b = i * 2 + p
            q = 1 - p
            drain(p)
            load_idx(b + 1, q)
            fire(q)
            xl_rows = xlb[p]
            xr_rows = xrb[p]

            def edge(e, ecarry):
                wrow = deg_col
                for h in range(HL):
                    xlv = xl_rows[e, pl.ds(h * C, C)]
                    xrv = xr_rows[e, pl.ds(h * C, C)]
                    s = xlv + xrv
                    lk = jnp.maximum(s, s * _f32(0.2))
                    logit = jnp.sum(lk * att_hv[h])
                    wv = jnp.exp(jnp.broadcast_to(logit, (16,)))
                    msg_buf[e, pl.ds(h * C, C)] = xlv * wv
                    wrow = jnp.where(lane_is[h], wv, wrow)
                dbuf[e, :] = wrow
                return ecarry

            lax.fori_loop(0, K, edge, 0)
            pltpu.sync_copy(msg_buf, s_sh.at[dstb[p]], add=True)
            pltpu.sync_copy(dbuf, d_sh.at[dstb[p]], add=True)
        return carry

    lax.fori_loop(0, NB // 2, pair, 0)
    drain(0)
    plsc.subcore_barrier()
    _copy_out(s_sh, s_out, cid, base)
    _copy_out(d_sh, d_out, cid, base)


_edge_call = functools.partial(
    pl.kernel,
    out_type=[jax.ShapeDtypeStruct((2, NP, FL), _f32),
              jax.ShapeDtypeStruct((2, NP, 16), _f32)],
    mesh=_MESH,
    compiler_params=_SC_PARAMS,
    scratch_types=[
        pltpu.VMEM((FL,), _f32),
        pltpu.VMEM((K,), _i32),
        pltpu.VMEM((K,), _i32),
        pltpu.VMEM((K,), _i32),
        pltpu.VMEM((K,), _i32),
        pltpu.VMEM((K, FL), _f32),
        pltpu.VMEM((K, FL), _f32),
        pltpu.VMEM((K, FL), _f32),
        pltpu.VMEM((K, FL), _f32),
        pltpu.VMEM((K, FL), _f32),
        pltpu.VMEM((K, 16), _f32),
        pltpu.VMEM_SHARED((NP, FL), _f32),
        pltpu.VMEM_SHARED((NP, 16), _f32),
        pltpu.SemaphoreType.DMA,
        pltpu.SemaphoreType.DMA,
        pltpu.SemaphoreType.DMA,
        pltpu.SemaphoreType.DMA,
    ],
)(_edge_pass)


def _gcn_pass(xgn_hbm, src_hbm, dst_hbm, o_out,
              src_idx0, src_idx1, dst_idx0, dst_idx1, rows0, rows1,
              o_sh, sem_a0, sem_a1):
    cid = lax.axis_index("c")
    sid = lax.axis_index("s")
    _zero_rows(rows0, FL)
    base = sid * RPT
    _zero_shared(rows0, o_sh, base)
    plsc.subcore_barrier()
    ebase = sid * EPT
    xg_t = xgn_hbm.at[cid]
    srcb = [src_idx0, src_idx1]
    dstb = [dst_idx0, dst_idx1]
    rowsb = [rows0, rows1]
    sems = [sem_a0, sem_a1]

    def load_idx(b, p):
        off = ebase + b * K
        pltpu.sync_copy(src_hbm.at[pl.ds(off, K)], srcb[p])
        pltpu.sync_copy(dst_hbm.at[pl.ds(off, K)], dstb[p])

    load_idx(0, 0)
    pltpu.async_copy(xg_t.at[srcb[0]], rowsb[0], sems[0])

    def pair(i, carry):
        for p in (0, 1):
            b = i * 2 + p
            q = 1 - p
            pltpu.make_async_copy(xg_t.at[srcb[p]], rowsb[p], sems[p]).wait()
            load_idx(b + 1, q)
            pltpu.async_copy(xg_t.at[srcb[q]], rowsb[q], sems[q])
            pltpu.sync_copy(rowsb[p], o_sh.at[dstb[p]], add=True)
        return carry

    lax.fori_loop(0, NB // 2, pair, 0)
    pltpu.make_async_copy(xg_t.at[srcb[0]], rowsb[0], sems[0]).wait()
    plsc.subcore_barrier()
    _copy_out(o_sh, o_out, cid, base)


_gcn_call = functools.partial(
    pl.kernel,
    out_type=jax.ShapeDtypeStruct((2, NP, FL), _f32),
    mesh=_MESH,
    compiler_params=_SC_PARAMS,
    scratch_types=[
        pltpu.VMEM((K,), _i32),
        pltpu.VMEM((K,), _i32),
        pltpu.VMEM((K,), _i32),
        pltpu.VMEM((K,), _i32),
        pltpu.VMEM((K, FL), _f32),
        pltpu.VMEM((K, FL), _f32),
        pltpu.VMEM_SHARED((NP, FL), _f32),
        pltpu.SemaphoreType.DMA,
        pltpu.SemaphoreType.DMA,
    ],
)(_gcn_pass)


def _mm2_body(x_ref, wl_ref, wr_ref, xl_ref, xr_ref):
    x = x_ref[...]
    xl = jnp.dot(x, wl_ref[...], preferred_element_type=_f32)
    xr = jnp.dot(x, wr_ref[...], preferred_element_type=_f32)
    xl_ref[0] = xl[:, :FL]
    xl_ref[1] = xl[:, FL:]
    xr_ref[0] = xr[:, :FL]
    xr_ref[1] = xr[:, FL:]


def _mid_body(s_ref, den_ref, deg_ref, erep_ref, bias1_ref, gamma_ref,
              beta_ref, pw_ref, wg_ref, xgn_ref, dinv_ref):
    S = s_ref[...]
    denom = den_ref[...]
    deg = deg_ref[...]
    den128 = jnp.dot(denom, erep_ref[...], preferred_element_type=_f32)
    h1 = S / den128 + bias1_ref[...]
    mu = jnp.mean(h1, axis=0, keepdims=True)
    xc = h1 - mu
    var = jnp.mean(xc * xc, axis=0, keepdims=True)
    hbn = xc * lax.rsqrt(var + 1e-5) * gamma_ref[...] + beta_ref[...]
    pw = pw_ref[0, 0]
    hp = jnp.where(hbn >= 0, hbn, pw * hbn)
    xg = jnp.dot(hp, wg_ref[...], preferred_element_type=_f32)
    dinv = lax.rsqrt(jnp.maximum(deg, 1.0))
    xgn = xg * dinv
    xgn_ref[0] = xgn[:, :FL]
    xgn_ref[1] = xgn[:, FL:]
    dinv_ref[...] = dinv


def _fin_body(o_ref, dinv_ref, bg_ref, out_ref):
    dinv = dinv_ref[...]
    bg = bg_ref[...]
    out_ref[:, :FL] = o_ref[0] * dinv + bg[:, :FL]
    out_ref[:, FL:] = o_ref[1] * dinv + bg[:, FL:]


def kernel(x, edge_index, Wl, Wr, att, bias1, gamma, beta, prelu_w, Wg, bias_g):
    xpad = jnp.pad(x, ((0, NP - N), (0, 0)))
    xlh, xrh = pl.pallas_call(
        _mm2_body,
        out_shape=[jax.ShapeDtypeStruct((2, NP, FL), _f32)] * 2,
    )(xpad, Wl, Wr)

    loops = jnp.arange(N, dtype=_i32)
    padi = jnp.full((EPAD - EFULL + K,), N, _i32)
    srcf = jnp.concatenate([edge_index[0], loops, padi])
    dstf = jnp.concatenate([edge_index[1], loops, padi])

    DEBUG_XLA_EDGE = False
    if DEBUG_XLA_EDGE:
        src0, dst0 = srcf[:EFULL], dstf[:EFULL]
        xlf = jnp.concatenate([xlh[0], xlh[1]], axis=1)[:N].reshape(N, H, C)
        xrf = jnp.concatenate([xrh[0], xrh[1]], axis=1)[:N].reshape(N, H, C)
        e = jax.nn.leaky_relu(xlf[src0] + xrf[dst0], negative_slope=0.2)
        logits = jnp.einsum('ehc,hc->eh', e, att)
        ex = jnp.exp(logits)
        denom = jax.ops.segment_sum(ex, dst0, num_segments=N)
        S = jax.ops.segment_sum(
            xlf[src0].reshape(EFULL, HC) * jnp.repeat(ex, C, axis=1),
            dst0, num_segments=N)
        deg = jax.ops.segment_sum(jnp.ones((EFULL, 1), _f32), dst0,
                                  num_segments=N)
    else:
        s_out, d_out = _edge_call(xlh, xrh, att.reshape(-1), srcf, dstf)
        S = jnp.concatenate([s_out[0, :N], s_out[1, :N]], axis=1)
        denom = jnp.concatenate([d_out[0, :N, 0:HL], d_out[1, :N, 0:HL]],
                                axis=1)
        deg = d_out[0, :N, HL:HL + 1]

    erep = jnp.kron(jnp.eye(H, dtype=_f32), jnp.ones((1, C), _f32))

    xgnh, dinv = pl.pallas_call(
        _mid_body,
        out_shape=[jax.ShapeDtypeStruct((2, N, FL), _f32),
                   jax.ShapeDtypeStruct((N, 1), _f32)],
    )(S, denom, deg, erep, bias1.reshape(1, -1), gamma.reshape(1, -1),
      beta.reshape(1, -1), prelu_w.reshape(1, 1), Wg)

    xgn_pad = jnp.pad(xgnh, ((0, 0), (0, NP - N), (0, 0)))
    o_out = _gcn_call(xgn_pad, srcf, dstf)

    out = pl.pallas_call(
        _fin_body,
        out_shape=jax.ShapeDtypeStruct((N, OUT), _f32),
    )(o_out[:, :N], dinv, bias_g.reshape(1, -1))
    return out


# speedup vs baseline: 4.1213x; 1.3966x over previous
import functools

import jax
import jax.numpy as jnp
from jax import lax
from jax.experimental import pallas as pl
from jax.experimental.pallas import tpu as pltpu
from jax.experimental.pallas import tpu_sc as plsc

N = 10000
NP = 10112
E = 320000
EFULL = E + N
K = 128
NB = 162
EPT = K * NB
EPAD = EPT * 16
H = 8
HL = 4
C = 16
HC = H * C
FL = HL * C
OUT = 128
RPT = NP // 16

_f32 = jnp.float32
_i32 = jnp.int32

_MESH = plsc.VectorSubcoreMesh(
    core_axis_name="c", subcore_axis_name="s", num_cores=2, num_subcores=16)
_SC_PARAMS = pltpu.CompilerParams(
    needs_layout_passes=False, use_tc_tiling_on_sc=False)


def _zero_rows(buf, ncols):
    def body(r, carry):
        for j in range(ncols // 16):
            buf[r, pl.ds(16 * j, 16)] = jnp.zeros((16,), _f32)
        return carry
    lax.fori_loop(0, K, body, 0)


def _zero_shared(src_buf, shared, base):
    for kk in range((RPT + K - 1) // K):
        nrows = min(K, RPT - kk * K)
        pltpu.sync_copy(src_buf.at[pl.ds(0, nrows)],
                        shared.at[pl.ds(base + kk * K, nrows)])


def _copy_out(shared, out_hbm, cid, base):
    for kk in range((RPT + K - 1) // K):
        nrows = min(K, RPT - kk * K)
        pltpu.sync_copy(shared.at[pl.ds(base + kk * K, nrows)],
                        out_hbm.at[cid, pl.ds(base + kk * K, nrows)])


def _edge_pass(xlh_hbm, xrh_hbm, att_hbm, src_hbm, dst_hbm, s_out, d_out,
               att_v, src_idx0, src_idx1, dst_idx0, dst_idx1,
               xl0, xl1, xr0, xr1, msg_buf, dbuf,
               s_sh, d_sh, sem_l0, sem_l1, sem_r0, sem_r1):
    cid = lax.axis_index("c")
    sid = lax.axis_index("s")
    pltpu.sync_copy(att_hbm.at[pl.ds(cid * FL, FL)], att_v)

    _zero_rows(xl0, FL)
    _zero_rows(dbuf, 16)
    base = sid * RPT
    _zero_shared(xl0, s_sh, base)
    _zero_shared(dbuf, d_sh, base)
    plsc.subcore_barrier()

    ebase = sid * EPT
    iot = lax.iota(_i32, 16)
    xl_t = xlh_hbm.at[cid]
    xr_t = xrh_hbm.at[cid]
    srcb = [src_idx0, src_idx1]
    dstb = [dst_idx0, dst_idx1]
    xlb = [xl0, xl1]
    xrb = [xr0, xr1]
    sl = [sem_l0, sem_l1]
    sr = [sem_r0, sem_r1]
    att_hv = [att_v[pl.ds(h * C, C)] for h in range(HL)]
    lane_is = [iot == h for h in range(HL)]
    deg_col = jnp.where(iot == HL, jnp.ones((16,), _f32),
                        jnp.zeros((16,), _f32))

    def load_idx(b, p):
        off = ebase + b * K
        pltpu.sync_copy(src_hbm.at[pl.ds(off, K)], srcb[p])
        pltpu.sync_copy(dst_hbm.at[pl.ds(off, K)], dstb[p])

    def fire(p):
        pltpu.async_copy(xl_t.at[srcb[p]], xlb[p], sl[p])
        pltpu.async_copy(xr_t.at[dstb[p]], xrb[p], sr[p])

    def drain(p):
        pltpu.make_async_copy(xl_t.at[srcb[p]], xlb[p], sl[p]).wait()
        pltpu.make_async_copy(xr_t.at[dstb[p]], xrb[p], sr[p]).wait()

    load_idx(0, 0)
    fire(0)

    def pair(i, carry):
        for p in (0, 1):
            b = i * 2 + p
            q = 1 - p
            drain(p)
            load_idx(b + 1, q)
            fire(q)
            xl_rows = xlb[p]
            xr_rows = xrb[p]

            @plsc.parallel_loop(0, K, unroll=4)
            def edge(e):
                wrow = deg_col
                for h in range(HL):
                    xlv = xl_rows[e, pl.ds(h * C, C)]
                    xrv = xr_rows[e, pl.ds(h * C, C)]
                    s = xlv + xrv
                    lk = jnp.maximum(s, s * _f32(0.2))
                    logit = jnp.sum(lk * att_hv[h])
                    wv = jnp.exp(jnp.broadcast_to(logit, (16,)))
                    msg_buf[e, pl.ds(h * C, C)] = xlv * wv
                    wrow = jnp.where(lane_is[h], wv, wrow)
                dbuf[e, :] = wrow
            pltpu.sync_copy(msg_buf, s_sh.at[dstb[p]], add=True)
            pltpu.sync_copy(dbuf, d_sh.at[dstb[p]], add=True)
        return carry

    lax.fori_loop(0, NB // 2, pair, 0)
    drain(0)
    plsc.subcore_barrier()
    _copy_out(s_sh, s_out, cid, base)
    _copy_out(d_sh, d_out, cid, base)


_edge_call = functools.partial(
    pl.kernel,
    out_type=[jax.ShapeDtypeStruct((2, NP, FL), _f32),
              jax.ShapeDtypeStruct((2, NP, 16), _f32)],
    mesh=_MESH,
    compiler_params=_SC_PARAMS,
    scratch_types=[
        pltpu.VMEM((FL,), _f32),
        pltpu.VMEM((K,), _i32),
        pltpu.VMEM((K,), _i32),
        pltpu.VMEM((K,), _i32),
        pltpu.VMEM((K,), _i32),
        pltpu.VMEM((K, FL), _f32),
        pltpu.VMEM((K, FL), _f32),
        pltpu.VMEM((K, FL), _f32),
        pltpu.VMEM((K, FL), _f32),
        pltpu.VMEM((K, FL), _f32),
        pltpu.VMEM((K, 16), _f32),
        pltpu.VMEM_SHARED((NP, FL), _f32),
        pltpu.VMEM_SHARED((NP, 16), _f32),
        pltpu.SemaphoreType.DMA,
        pltpu.SemaphoreType.DMA,
        pltpu.SemaphoreType.DMA,
        pltpu.SemaphoreType.DMA,
    ],
)(_edge_pass)


def _gcn_pass(xgn_hbm, src_hbm, dst_hbm, o_out,
              src_idx0, src_idx1, dst_idx0, dst_idx1, rows0, rows1,
              o_sh, sem_a0, sem_a1):
    cid = lax.axis_index("c")
    sid = lax.axis_index("s")
    _zero_rows(rows0, FL)
    base = sid * RPT
    _zero_shared(rows0, o_sh, base)
    plsc.subcore_barrier()
    ebase = sid * EPT
    xg_t = xgn_hbm.at[cid]
    srcb = [src_idx0, src_idx1]
    dstb = [dst_idx0, dst_idx1]
    rowsb = [rows0, rows1]
    sems = [sem_a0, sem_a1]

    def load_idx(b, p):
        off = ebase + b * K
        pltpu.sync_copy(src_hbm.at[pl.ds(off, K)], srcb[p])
        pltpu.sync_copy(dst_hbm.at[pl.ds(off, K)], dstb[p])

    load_idx(0, 0)
    pltpu.async_copy(xg_t.at[srcb[0]], rowsb[0], sems[0])

    def pair(i, carry):
        for p in (0, 1):
            b = i * 2 + p
            q = 1 - p
            pltpu.make_async_copy(xg_t.at[srcb[p]], rowsb[p], sems[p]).wait()
            load_idx(b + 1, q)
            pltpu.async_copy(xg_t.at[srcb[q]], rowsb[q], sems[q])
            pltpu.sync_copy(rowsb[p], o_sh.at[dstb[p]], add=True)
        return carry

    lax.fori_loop(0, NB // 2, pair, 0)
    pltpu.make_async_copy(xg_t.at[srcb[0]], rowsb[0], sems[0]).wait()
    plsc.subcore_barrier()
    _copy_out(o_sh, o_out, cid, base)


_gcn_call = functools.partial(
    pl.kernel,
    out_type=jax.ShapeDtypeStruct((2, NP, FL), _f32),
    mesh=_MESH,
    compiler_params=_SC_PARAMS,
    scratch_types=[
        pltpu.VMEM((K,), _i32),
        pltpu.VMEM((K,), _i32),
        pltpu.VMEM((K,), _i32),
        pltpu.VMEM((K,), _i32),
        pltpu.VMEM((K, FL), _f32),
        pltpu.VMEM((K, FL), _f32),
        pltpu.VMEM_SHARED((NP, FL), _f32),
        pltpu.SemaphoreType.DMA,
        pltpu.SemaphoreType.DMA,
    ],
)(_gcn_pass)


def _mm2_body(x_ref, wl_ref, wr_ref, xl_ref, xr_ref):
    x = x_ref[...]
    xl = jnp.dot(x, wl_ref[...], preferred_element_type=_f32)
    xr = jnp.dot(x, wr_ref[...], preferred_element_type=_f32)
    xl_ref[0] = xl[:, :FL]
    xl_ref[1] = xl[:, FL:]
    xr_ref[0] = xr[:, :FL]
    xr_ref[1] = xr[:, FL:]


def _mid_body(s_ref, den_ref, deg_ref, erep_ref, bias1_ref, gamma_ref,
              beta_ref, pw_ref, wg_ref, xgn_ref, dinv_ref):
    S = s_ref[...]
    denom = den_ref[...]
    deg = deg_ref[...]
    den128 = jnp.dot(denom, erep_ref[...], preferred_element_type=_f32)
    h1 = S / den128 + bias1_ref[...]
    mu = jnp.mean(h1, axis=0, keepdims=True)
    xc = h1 - mu
    var = jnp.mean(xc * xc, axis=0, keepdims=True)
    hbn = xc * lax.rsqrt(var + 1e-5) * gamma_ref[...] + beta_ref[...]
    pw = pw_ref[0, 0]
    hp = jnp.where(hbn >= 0, hbn, pw * hbn)
    xg = jnp.dot(hp, wg_ref[...], preferred_element_type=_f32)
    dinv = lax.rsqrt(jnp.maximum(deg, 1.0))
    xgn = xg * dinv
    xgn_ref[0] = xgn[:, :FL]
    xgn_ref[1] = xgn[:, FL:]
    dinv_ref[...] = dinv


def _fin_body(o_ref, dinv_ref, bg_ref, out_ref):
    dinv = dinv_ref[...]
    bg = bg_ref[...]
    out_ref[:, :FL] = o_ref[0] * dinv + bg[:, :FL]
    out_ref[:, FL:] = o_ref[1] * dinv + bg[:, FL:]


def kernel(x, edge_index, Wl, Wr, att, bias1, gamma, beta, prelu_w, Wg, bias_g):
    xpad = jnp.pad(x, ((0, NP - N), (0, 0)))
    xlh, xrh = pl.pallas_call(
        _mm2_body,
        out_shape=[jax.ShapeDtypeStruct((2, NP, FL), _f32)] * 2,
    )(xpad, Wl, Wr)

    loops = jnp.arange(N, dtype=_i32)
    padi = jnp.full((EPAD - EFULL + K,), N, _i32)
    srcf = jnp.concatenate([edge_index[0], loops, padi])
    dstf = jnp.concatenate([edge_index[1], loops, padi])

    DEBUG_XLA_EDGE = False
    if DEBUG_XLA_EDGE:
        src0, dst0 = srcf[:EFULL], dstf[:EFULL]
        xlf = jnp.concatenate([xlh[0], xlh[1]], axis=1)[:N].reshape(N, H, C)
        xrf = jnp.concatenate([xrh[0], xrh[1]], axis=1)[:N].reshape(N, H, C)
        e = jax.nn.leaky_relu(xlf[src0] + xrf[dst0], negative_slope=0.2)
        logits = jnp.einsum('ehc,hc->eh', e, att)
        ex = jnp.exp(logits)
        denom = jax.ops.segment_sum(ex, dst0, num_segments=N)
        S = jax.ops.segment_sum(
            xlf[src0].reshape(EFULL, HC) * jnp.repeat(ex, C, axis=1),
            dst0, num_segments=N)
        deg = jax.ops.segment_sum(jnp.ones((EFULL, 1), _f32), dst0,
                                  num_segments=N)
    else:
        s_out, d_out = _edge_call(xlh, xrh, att.reshape(-1), srcf, dstf)
        S = jnp.concatenate([s_out[0, :N], s_out[1, :N]], axis=1)
        denom = jnp.concatenate([d_out[0, :N, 0:HL], d_out[1, :N, 0:HL]],
                                axis=1)
        deg = d_out[0, :N, HL:HL + 1]

    erep = jnp.kron(jnp.eye(H, dtype=_f32), jnp.ones((1, C), _f32))

    xgnh, dinv = pl.pallas_call(
        _mid_body,
        out_shape=[jax.ShapeDtypeStruct((2, N, FL), _f32),
                   jax.ShapeDtypeStruct((N, 1), _f32)],
    )(S, denom, deg, erep, bias1.reshape(1, -1), gamma.reshape(1, -1),
      beta.reshape(1, -1), prelu_w.reshape(1, 1), Wg)

    xgn_pad = jnp.pad(xgnh, ((0, 0), (0, NP - N), (0, 0)))
    o_out = _gcn_call(xgn_pad, srcf, dstf)

    out = pl.pallas_call(
        _fin_body,
        out_shape=jax.ShapeDtypeStruct((N, OUT), _f32),
    )(o_out[:, :N], dinv, bias_g.reshape(1, -1))
    return out


# speedup vs baseline: 4.1279x; 1.0016x over previous
import functools

import jax
import jax.numpy as jnp
from jax import lax
from jax.experimental import pallas as pl
from jax.experimental.pallas import tpu as pltpu
from jax.experimental.pallas import tpu_sc as plsc

N = 10000
NP = 10112
E = 320000
EFULL = E + N
K = 128
NB = 162
EPT = K * NB
EPAD = EPT * 16
H = 8
HL = 4
C = 16
HC = H * C
FL = HL * C
OUT = 128
RPT = NP // 16

_f32 = jnp.float32
_i32 = jnp.int32

_MESH = plsc.VectorSubcoreMesh(
    core_axis_name="c", subcore_axis_name="s", num_cores=2, num_subcores=16)
_SC_PARAMS = pltpu.CompilerParams(
    needs_layout_passes=False, use_tc_tiling_on_sc=False)


def _zero_rows(buf, ncols):
    def body(r, carry):
        for j in range(ncols // 16):
            buf[r, pl.ds(16 * j, 16)] = jnp.zeros((16,), _f32)
        return carry
    lax.fori_loop(0, K, body, 0)


def _zero_shared(src_buf, shared, base):
    for kk in range((RPT + K - 1) // K):
        nrows = min(K, RPT - kk * K)
        pltpu.sync_copy(src_buf.at[pl.ds(0, nrows)],
                        shared.at[pl.ds(base + kk * K, nrows)])


def _copy_out(shared, out_hbm, cid, base):
    for kk in range((RPT + K - 1) // K):
        nrows = min(K, RPT - kk * K)
        pltpu.sync_copy(shared.at[pl.ds(base + kk * K, nrows)],
                        out_hbm.at[cid, pl.ds(base + kk * K, nrows)])


def _edge_pass(xlh_hbm, xrh_hbm, att_hbm, src_hbm, dst_hbm, s_out, d_out,
               att_v, src_idx0, src_idx1, dst_idx0, dst_idx1,
               xl0, xl1, xr0, xr1, msg_buf, dbuf,
               s_sh, d_sh, sem_l0, sem_l1, sem_r0, sem_r1):
    cid = lax.axis_index("c")
    sid = lax.axis_index("s")
    pltpu.sync_copy(att_hbm.at[pl.ds(cid * FL, FL)], att_v)

    _zero_rows(xl0, FL)
    _zero_rows(dbuf, 16)
    base = sid * RPT
    _zero_shared(xl0, s_sh, base)
    _zero_shared(dbuf, d_sh, base)
    plsc.subcore_barrier()

    ebase = sid * EPT
    iot = lax.iota(_i32, 16)
    xl_t = xlh_hbm.at[cid]
    xr_t = xrh_hbm.at[cid]
    srcb = [src_idx0, src_idx1]
    dstb = [dst_idx0, dst_idx1]
    xlb = [xl0, xl1]
    xrb = [xr0, xr1]
    sl = [sem_l0, sem_l1]
    sr = [sem_r0, sem_r1]
    att_hv = [att_v[pl.ds(h * C, C)] for h in range(HL)]
    lane_is = [iot == h for h in range(HL)]
    deg_col = jnp.where(iot == HL, jnp.ones((16,), _f32),
                        jnp.zeros((16,), _f32))

    def load_idx(b, p):
        off = ebase + b * K
        pltpu.sync_copy(src_hbm.at[pl.ds(off, K)], srcb[p])
        pltpu.sync_copy(dst_hbm.at[pl.ds(off, K)], dstb[p])

    def fire(p):
        pltpu.async_copy(xl_t.at[srcb[p]], xlb[p], sl[p])
        pltpu.async_copy(xr_t.at[dstb[p]], xrb[p], sr[p])

    def drain(p):
        pltpu.make_async_copy(xl_t.at[srcb[p]], xlb[p], sl[p]).wait()
        pltpu.make_async_copy(xr_t.at[dstb[p]], xrb[p], sr[p]).wait()

    load_idx(0, 0)
    fire(0)

    def pair(i, carry):
        for p in (0, 1):
            b = i * 2 + p
            q = 1 - p
            drain(p)
            load_idx(b + 1, q)
            fire(q)
            xl_rows = xlb[p]
            xr_rows = xrb[p]

            @plsc.parallel_loop(0, K, unroll=8)
            def edge(e):
                wrow = deg_col
                for h in range(HL):
                    xlv = xl_rows[e, pl.ds(h * C, C)]
                    xrv = xr_rows[e, pl.ds(h * C, C)]
                    s = xlv + xrv
                    lk = jnp.maximum(s, s * _f32(0.2))
                    logit = jnp.sum(lk * att_hv[h])
                    wv = jnp.exp(jnp.broadcast_to(logit, (16,)))
                    msg_buf[e, pl.ds(h * C, C)] = xlv * wv
                    wrow = jnp.where(lane_is[h], wv, wrow)
                dbuf[e, :] = wrow
            pltpu.sync_copy(msg_buf, s_sh.at[dstb[p]], add=True)
            pltpu.sync_copy(dbuf, d_sh.at[dstb[p]], add=True)
        return carry

    lax.fori_loop(0, NB // 2, pair, 0)
    drain(0)
    plsc.subcore_barrier()
    _copy_out(s_sh, s_out, cid, base)
    _copy_out(d_sh, d_out, cid, base)


_edge_call = functools.partial(
    pl.kernel,
    out_type=[jax.ShapeDtypeStruct((2, NP, FL), _f32),
              jax.ShapeDtypeStruct((2, NP, 16), _f32)],
    mesh=_MESH,
    compiler_params=_SC_PARAMS,
    scratch_types=[
        pltpu.VMEM((FL,), _f32),
        pltpu.VMEM((K,), _i32),
        pltpu.VMEM((K,), _i32),
        pltpu.VMEM((K,), _i32),
        pltpu.VMEM((K,), _i32),
        pltpu.VMEM((K, FL), _f32),
        pltpu.VMEM((K, FL), _f32),
        pltpu.VMEM((K, FL), _f32),
        pltpu.VMEM((K, FL), _f32),
        pltpu.VMEM((K, FL), _f32),
        pltpu.VMEM((K, 16), _f32),
        pltpu.VMEM_SHARED((NP, FL), _f32),
        pltpu.VMEM_SHARED((NP, 16), _f32),
        pltpu.SemaphoreType.DMA,
        pltpu.SemaphoreType.DMA,
        pltpu.SemaphoreType.DMA,
        pltpu.SemaphoreType.DMA,
    ],
)(_edge_pass)


def _gcn_pass(xgn_hbm, src_hbm, dst_hbm, o_out,
              src_idx0, src_idx1, dst_idx0, dst_idx1, rows0, rows1,
              o_sh, sem_a0, sem_a1):
    cid = lax.axis_index("c")
    sid = lax.axis_index("s")
    _zero_rows(rows0, FL)
    base = sid * RPT
    _zero_shared(rows0, o_sh, base)
    plsc.subcore_barrier()
    ebase = sid * EPT
    xg_t = xgn_hbm.at[cid]
    srcb = [src_idx0, src_idx1]
    dstb = [dst_idx0, dst_idx1]
    rowsb = [rows0, rows1]
    sems = [sem_a0, sem_a1]

    def load_idx(b, p):
        off = ebase + b * K
        pltpu.sync_copy(src_hbm.at[pl.ds(off, K)], srcb[p])
        pltpu.sync_copy(dst_hbm.at[pl.ds(off, K)], dstb[p])

    load_idx(0, 0)
    pltpu.async_copy(xg_t.at[srcb[0]], rowsb[0], sems[0])

    def pair(i, carry):
        for p in (0, 1):
            b = i * 2 + p
            q = 1 - p
            pltpu.make_async_copy(xg_t.at[srcb[p]], rowsb[p], sems[p]).wait()
            load_idx(b + 1, q)
            pltpu.async_copy(xg_t.at[srcb[q]], rowsb[q], sems[q])
            pltpu.sync_copy(rowsb[p], o_sh.at[dstb[p]], add=True)
        return carry

    lax.fori_loop(0, NB // 2, pair, 0)
    pltpu.make_async_copy(xg_t.at[srcb[0]], rowsb[0], sems[0]).wait()
    plsc.subcore_barrier()
    _copy_out(o_sh, o_out, cid, base)


_gcn_call = functools.partial(
    pl.kernel,
    out_type=jax.ShapeDtypeStruct((2, NP, FL), _f32),
    mesh=_MESH,
    compiler_params=_SC_PARAMS,
    scratch_types=[
        pltpu.VMEM((K,), _i32),
        pltpu.VMEM((K,), _i32),
        pltpu.VMEM((K,), _i32),
        pltpu.VMEM((K,), _i32),
        pltpu.VMEM((K, FL), _f32),
        pltpu.VMEM((K, FL), _f32),
        pltpu.VMEM_SHARED((NP, FL), _f32),
        pltpu.SemaphoreType.DMA,
        pltpu.SemaphoreType.DMA,
    ],
)(_gcn_pass)


def _mm2_body(x_ref, wl_ref, wr_ref, xl_ref, xr_ref):
    x = x_ref[...]
    xl = jnp.dot(x, wl_ref[...], preferred_element_type=_f32)
    xr = jnp.dot(x, wr_ref[...], preferred_element_type=_f32)
    xl_ref[0] = xl[:, :FL]
    xl_ref[1] = xl[:, FL:]
    xr_ref[0] = xr[:, :FL]
    xr_ref[1] = xr[:, FL:]


def _mid_body(s_ref, den_ref, deg_ref, erep_ref, bias1_ref, gamma_ref,
              beta_ref, pw_ref, wg_ref, xgn_ref, dinv_ref):
    S = s_ref[...]
    denom = den_ref[...]
    deg = deg_ref[...]
    den128 = jnp.dot(denom, erep_ref[...], preferred_element_type=_f32)
    h1 = S / den128 + bias1_ref[...]
    mu = jnp.mean(h1, axis=0, keepdims=True)
    xc = h1 - mu
    var = jnp.mean(xc * xc, axis=0, keepdims=True)
    hbn = xc * lax.rsqrt(var + 1e-5) * gamma_ref[...] + beta_ref[...]
    pw = pw_ref[0, 0]
    hp = jnp.where(hbn >= 0, hbn, pw * hbn)
    xg = jnp.dot(hp, wg_ref[...], preferred_element_type=_f32)
    dinv = lax.rsqrt(jnp.maximum(deg, 1.0))
    xgn = xg * dinv
    xgn_ref[0] = xgn[:, :FL]
    xgn_ref[1] = xgn[:, FL:]
    dinv_ref[...] = dinv


def _fin_body(o_ref, dinv_ref, bg_ref, out_ref):
    dinv = dinv_ref[...]
    bg = bg_ref[...]
    out_ref[:, :FL] = o_ref[0] * dinv + bg[:, :FL]
    out_ref[:, FL:] = o_ref[1] * dinv + bg[:, FL:]


def kernel(x, edge_index, Wl, Wr, att, bias1, gamma, beta, prelu_w, Wg, bias_g):
    xpad = jnp.pad(x, ((0, NP - N), (0, 0)))
    xlh, xrh = pl.pallas_call(
        _mm2_body,
        out_shape=[jax.ShapeDtypeStruct((2, NP, FL), _f32)] * 2,
    )(xpad, Wl, Wr)

    loops = jnp.arange(N, dtype=_i32)
    padi = jnp.full((EPAD - EFULL + K,), N, _i32)
    srcf = jnp.concatenate([edge_index[0], loops, padi])
    dstf = jnp.concatenate([edge_index[1], loops, padi])

    DEBUG_XLA_EDGE = False
    if DEBUG_XLA_EDGE:
        src0, dst0 = srcf[:EFULL], dstf[:EFULL]
        xlf = jnp.concatenate([xlh[0], xlh[1]], axis=1)[:N].reshape(N, H, C)
        xrf = jnp.concatenate([xrh[0], xrh[1]], axis=1)[:N].reshape(N, H, C)
        e = jax.nn.leaky_relu(xlf[src0] + xrf[dst0], negative_slope=0.2)
        logits = jnp.einsum('ehc,hc->eh', e, att)
        ex = jnp.exp(logits)
        denom = jax.ops.segment_sum(ex, dst0, num_segments=N)
        S = jax.ops.segment_sum(
            xlf[src0].reshape(EFULL, HC) * jnp.repeat(ex, C, axis=1),
            dst0, num_segments=N)
        deg = jax.ops.segment_sum(jnp.ones((EFULL, 1), _f32), dst0,
                                  num_segments=N)
    else:
        s_out, d_out = _edge_call(xlh, xrh, att.reshape(-1), srcf, dstf)
        S = jnp.concatenate([s_out[0, :N], s_out[1, :N]], axis=1)
        denom = jnp.concatenate([d_out[0, :N, 0:HL], d_out[1, :N, 0:HL]],
                                axis=1)
        deg = d_out[0, :N, HL:HL + 1]

    erep = jnp.kron(jnp.eye(H, dtype=_f32), jnp.ones((1, C), _f32))

    xgnh, dinv = pl.pallas_call(
        _mid_body,
        out_shape=[jax.ShapeDtypeStruct((2, N, FL), _f32),
                   jax.ShapeDtypeStruct((N, 1), _f32)],
    )(S, denom, deg, erep, bias1.reshape(1, -1), gamma.reshape(1, -1),
      beta.reshape(1, -1), prelu_w.reshape(1, 1), Wg)

    xgn_pad = jnp.pad(xgnh, ((0, 0), (0, NP - N), (0, 0)))
    o_out = _gcn_call(xgn_pad, srcf, dstf)

    out = pl.pallas_call(
        _fin_body,
        out_shape=jax.ShapeDtypeStruct((N, OUT), _f32),
    )(o_out[:, :N], dinv, bias_g.reshape(1, -1))
    return out


# speedup vs baseline: 4.2535x; 1.0304x over previous
import functools

import jax
import jax.numpy as jnp
from jax import lax
from jax.experimental import pallas as pl
from jax.experimental.pallas import tpu as pltpu
from jax.experimental.pallas import tpu_sc as plsc

N = 10000
NP = 10112
E = 320000
EFULL = E + N
K = 128
NB = 162
EPT = K * NB
EPAD = EPT * 16
H = 8
HL = 4
C = 16
HC = H * C
FL = HL * C
OUT = 128
RPT = NP // 16

_f32 = jnp.float32
_i32 = jnp.int32

_MESH = plsc.VectorSubcoreMesh(
    core_axis_name="c", subcore_axis_name="s", num_cores=2, num_subcores=16)
_SC_PARAMS = pltpu.CompilerParams(
    needs_layout_passes=False, use_tc_tiling_on_sc=False)


def _zero_rows(buf, ncols):
    def body(r, carry):
        for j in range(ncols // 16):
            buf[r, pl.ds(16 * j, 16)] = jnp.zeros((16,), _f32)
        return carry
    lax.fori_loop(0, K, body, 0)


def _zero_shared(src_buf, shared, base):
    for kk in range((RPT + K - 1) // K):
        nrows = min(K, RPT - kk * K)
        pltpu.sync_copy(src_buf.at[pl.ds(0, nrows)],
                        shared.at[pl.ds(base + kk * K, nrows)])


def _copy_out(shared, out_hbm, cid, base):
    for kk in range((RPT + K - 1) // K):
        nrows = min(K, RPT - kk * K)
        pltpu.sync_copy(shared.at[pl.ds(base + kk * K, nrows)],
                        out_hbm.at[cid, pl.ds(base + kk * K, nrows)])


def _edge_pass(xlh_hbm, xrh_hbm, att_hbm, src_hbm, dst_hbm, s_out, d_out,
               att_v, src_idx0, src_idx1, dst_idx0, dst_idx1,
               xl0, xl1, xr0, xr1, msg_buf, dbuf,
               s_sh, d_sh, sem_l0, sem_l1, sem_r0, sem_r1):
    cid = lax.axis_index("c")
    sid = lax.axis_index("s")
    pltpu.sync_copy(att_hbm.at[pl.ds(cid * FL, FL)], att_v)

    _zero_rows(xl0, FL)
    _zero_rows(dbuf, 16)
    base = sid * RPT
    _zero_shared(xl0, s_sh, base)
    _zero_shared(dbuf, d_sh, base)
    plsc.subcore_barrier()

    ebase = sid * EPT
    iot = lax.iota(_i32, 16)
    xl_t = xlh_hbm.at[cid]
    xr_t = xrh_hbm.at[cid]
    srcb = [src_idx0, src_idx1]
    dstb = [dst_idx0, dst_idx1]
    xlb = [xl0, xl1]
    xrb = [xr0, xr1]
    sl = [sem_l0, sem_l1]
    sr = [sem_r0, sem_r1]
    att_hv = [att_v[pl.ds(h * C, C)] for h in range(HL)]
    lane_is = [iot == h for h in range(HL)]
    deg_col = jnp.where(iot == HL, jnp.ones((16,), _f32),
                        jnp.zeros((16,), _f32))

    def load_idx(b, p):
        off = ebase + b * K
        pltpu.sync_copy(src_hbm.at[pl.ds(off, K)], srcb[p])
        pltpu.sync_copy(dst_hbm.at[pl.ds(off, K)], dstb[p])

    def fire(p):
        pltpu.async_copy(xl_t.at[srcb[p]], xlb[p], sl[p])
        pltpu.async_copy(xr_t.at[dstb[p]], xrb[p], sr[p])

    def drain(p):
        pltpu.make_async_copy(xl_t.at[srcb[p]], xlb[p], sl[p]).wait()
        pltpu.make_async_copy(xr_t.at[dstb[p]], xrb[p], sr[p]).wait()

    load_idx(0, 0)
    fire(0)

    def pair(i, carry):
        for p in (0, 1):
            b = i * 2 + p
            q = 1 - p
            drain(p)
            load_idx(b + 1, q)
            fire(q)
            xl_rows = xlb[p]
            xr_rows = xrb[p]

            @plsc.parallel_loop(0, K, unroll=4)
            def edge(e):
                wrow = deg_col
                for h in range(HL):
                    xlv = xl_rows[e, pl.ds(h * C, C)]
                    xrv = xr_rows[e, pl.ds(h * C, C)]
                    s = xlv + xrv
                    lk = jnp.maximum(s, s * _f32(0.2))
                    logit = jnp.sum(lk * att_hv[h])
                    wv = jnp.exp(jnp.broadcast_to(logit, (16,)))
                    msg_buf[e, pl.ds(h * C, C)] = xlv * wv
                    wrow = jnp.where(lane_is[h], wv, wrow)
                dbuf[e, :] = wrow
            pltpu.sync_copy(msg_buf, s_sh.at[dstb[p]], add=True)
            pltpu.sync_copy(dbuf, d_sh.at[dstb[p]], add=True)
        return carry

    lax.fori_loop(0, NB // 2, pair, 0)
    drain(0)
    plsc.subcore_barrier()
    _copy_out(s_sh, s_out, cid, base)
    _copy_out(d_sh, d_out, cid, base)


_edge_call = functools.partial(
    pl.kernel,
    out_type=[jax.ShapeDtypeStruct((2, NP, FL), _f32),
              jax.ShapeDtypeStruct((2, NP, 16), _f32)],
    mesh=_MESH,
    compiler_params=_SC_PARAMS,
    scratch_types=[
        pltpu.VMEM((FL,), _f32),
        pltpu.VMEM((K,), _i32),
        pltpu.VMEM((K,), _i32),
        pltpu.VMEM((K,), _i32),
        pltpu.VMEM((K,), _i32),
        pltpu.VMEM((K, FL), _f32),
        pltpu.VMEM((K, FL), _f32),
        pltpu.VMEM((K, FL), _f32),
        pltpu.VMEM((K, FL), _f32),
        pltpu.VMEM((K, FL), _f32),
        pltpu.VMEM((K, 16), _f32),
        pltpu.VMEM_SHARED((NP, FL), _f32),
        pltpu.VMEM_SHARED((NP, 16), _f32),
        pltpu.SemaphoreType.DMA,
        pltpu.SemaphoreType.DMA,
        pltpu.SemaphoreType.DMA,
        pltpu.SemaphoreType.DMA,
    ],
)(_edge_pass)


def _gcn_pass(xgn_hbm, src_hbm, dst_hbm, o_out,
              src_idx0, src_idx1, dst_idx0, dst_idx1, rows0, rows1,
              o_sh, sem_a0, sem_a1):
    cid = lax.axis_index("c")
    sid = lax.axis_index("s")
    _zero_rows(rows0, FL)
    base = sid * RPT
    _zero_shared(rows0, o_sh, base)
    plsc.subcore_barrier()
    ebase = sid * EPT
    xg_t = xgn_hbm.at[cid]
    srcb = [src_idx0, src_idx1]
    dstb = [dst_idx0, dst_idx1]
    rowsb = [rows0, rows1]
    sems = [sem_a0, sem_a1]

    def load_idx(b, p):
        off = ebase + b * K
        pltpu.sync_copy(src_hbm.at[pl.ds(off, K)], srcb[p])
        pltpu.sync_copy(dst_hbm.at[pl.ds(off, K)], dstb[p])

    load_idx(0, 0)
    pltpu.async_copy(xg_t.at[srcb[0]], rowsb[0], sems[0])

    def pair(i, carry):
        for p in (0, 1):
            b = i * 2 + p
            q = 1 - p
            pltpu.make_async_copy(xg_t.at[srcb[p]], rowsb[p], sems[p]).wait()
            load_idx(b + 1, q)
            pltpu.async_copy(xg_t.at[srcb[q]], rowsb[q], sems[q])
            pltpu.sync_copy(rowsb[p], o_sh.at[dstb[p]], add=True)
        return carry

    lax.fori_loop(0, NB // 2, pair, 0)
    pltpu.make_async_copy(xg_t.at[srcb[0]], rowsb[0], sems[0]).wait()
    plsc.subcore_barrier()
    _copy_out(o_sh, o_out, cid, base)


_gcn_call = functools.partial(
    pl.kernel,
    out_type=jax.ShapeDtypeStruct((2, NP, FL), _f32),
    mesh=_MESH,
    compiler_params=_SC_PARAMS,
    scratch_types=[
        pltpu.VMEM((K,), _i32),
        pltpu.VMEM((K,), _i32),
        pltpu.VMEM((K,), _i32),
        pltpu.VMEM((K,), _i32),
        pltpu.VMEM((K, FL), _f32),
        pltpu.VMEM((K, FL), _f32),
        pltpu.VMEM_SHARED((NP, FL), _f32),
        pltpu.SemaphoreType.DMA,
        pltpu.SemaphoreType.DMA,
    ],
)(_gcn_pass)


def _mm2_body(x_ref, wl_ref, wr_ref, xl_ref, xr_ref):
    x = x_ref[...]
    xl = jnp.dot(x, wl_ref[...], preferred_element_type=_f32)
    xr = jnp.dot(x, wr_ref[...], preferred_element_type=_f32)
    xl_ref[0] = xl[:, :FL]
    xl_ref[1] = xl[:, FL:]
    xr_ref[0] = xr[:, :FL]
    xr_ref[1] = xr[:, FL:]


def _mid_body(s_ref, d_ref, erep_ref, bias1_ref, gamma_ref,
              beta_ref, pw_ref, wg_ref, xgn_ref, dinv_ref):
    deg = d_ref[0, :N, HL:HL + 1]
    dinv = lax.rsqrt(jnp.maximum(deg, 1.0))
    pw = pw_ref[0, 0]
    erep = erep_ref[...]
    xg = None
    zpad = jnp.zeros((NP - N, FL), _f32)
    for i in range(2):
        denom = d_ref[i, :N, 0:HL]
        den64 = jnp.dot(denom, erep, preferred_element_type=_f32)
        h1 = s_ref[i, :N] / den64 + bias1_ref[i]
        mu = jnp.mean(h1, axis=0, keepdims=True)
        xc = h1 - mu
        var = jnp.mean(xc * xc, axis=0, keepdims=True)
        hbn = xc * lax.rsqrt(var + 1e-5) * gamma_ref[i] + beta_ref[i]
        hp = jnp.where(hbn >= 0, hbn, pw * hbn)
        part = jnp.dot(hp, wg_ref[pl.ds(i * FL, FL), :],
                       preferred_element_type=_f32)
        xg = part if xg is None else xg + part
    xgn = xg * dinv
    xgn_ref[0, :N] = xgn[:, :FL]
    xgn_ref[1, :N] = xgn[:, FL:]
    xgn_ref[0, N:] = zpad
    xgn_ref[1, N:] = zpad
    dinv_ref[...] = dinv


def _fin_body(o_ref, dinv_ref, bg_ref, out_ref):
    dinv = dinv_ref[...]
    bg = bg_ref[...]
    out_ref[:, :FL] = o_ref[0, :N] * dinv + bg[:, :FL]
    out_ref[:, FL:] = o_ref[1, :N] * dinv + bg[:, FL:]


def kernel(x, edge_index, Wl, Wr, att, bias1, gamma, beta, prelu_w, Wg, bias_g):
    xpad = jnp.pad(x, ((0, NP - N), (0, 0)))
    xlh, xrh = pl.pallas_call(
        _mm2_body,
        out_shape=[jax.ShapeDtypeStruct((2, NP, FL), _f32)] * 2,
    )(xpad, Wl, Wr)

    loops = jnp.arange(N, dtype=_i32)
    padi = jnp.full((EPAD - EFULL + K,), N, _i32)
    srcf = jnp.concatenate([edge_index[0], loops, padi])
    dstf = jnp.concatenate([edge_index[1], loops, padi])

    s_out, d_out = _edge_call(xlh, xrh, att.reshape(-1), srcf, dstf)

    erep = jnp.kron(jnp.eye(HL, dtype=_f32), jnp.ones((1, C), _f32))

    xgn_pad, dinv = pl.pallas_call(
        _mid_body,
        out_shape=[jax.ShapeDtypeStruct((2, NP, FL), _f32),
                   jax.ShapeDtypeStruct((N, 1), _f32)],
    )(s_out, d_out, erep, bias1.reshape(2, 1, FL), gamma.reshape(2, 1, FL),
      beta.reshape(2, 1, FL), prelu_w.reshape(1, 1), Wg)

    o_out = _gcn_call(xgn_pad, srcf, dstf)

    out = pl.pallas_call(
        _fin_body,
        out_shape=jax.ShapeDtypeStruct((N, OUT), _f32),
    )(o_out, dinv, bias_g.reshape(1, -1))
    return out


# speedup vs baseline: 4.7592x; 1.1189x over previous
import functools

import jax
import jax.numpy as jnp
from jax import lax
from jax.experimental import pallas as pl
from jax.experimental.pallas import tpu as pltpu
from jax.experimental.pallas import tpu_sc as plsc

N = 10000
NP = 10112
E = 320000
EFULL = E + N
K = 128
NB = 162
EPT = K * NB
EPAD = EPT * 16
H = 8
HL = 4
C = 16
HC = H * C
FL = HL * C
OUT = 128
RPT = NP // 16

_f32 = jnp.float32
_i32 = jnp.int32

_MESH = plsc.VectorSubcoreMesh(
    core_axis_name="c", subcore_axis_name="s", num_cores=2, num_subcores=16)
_SC_PARAMS = pltpu.CompilerParams(
    needs_layout_passes=False, use_tc_tiling_on_sc=False)


def _zero_rows(buf, ncols):
    def body(r, carry):
        for j in range(ncols // 16):
            buf[r, pl.ds(16 * j, 16)] = jnp.zeros((16,), _f32)
        return carry
    lax.fori_loop(0, K, body, 0)


def _zero_shared(src_buf, shared, base):
    for kk in range((RPT + K - 1) // K):
        nrows = min(K, RPT - kk * K)
        pltpu.sync_copy(src_buf.at[pl.ds(0, nrows)],
                        shared.at[pl.ds(base + kk * K, nrows)])


def _copy_out(shared, out_hbm, cid, base):
    for kk in range((RPT + K - 1) // K):
        nrows = min(K, RPT - kk * K)
        pltpu.sync_copy(shared.at[pl.ds(base + kk * K, nrows)],
                        out_hbm.at[cid, pl.ds(base + kk * K, nrows)])


def _edge_pass(xlh_hbm, xrh_hbm, att_hbm, src_hbm, dst_hbm, s_out, d_out,
               att_v, src_idx0, src_idx1, dst_idx0, dst_idx1,
               xl0, xl1, xr0, xr1, msg_buf, dbuf,
               s_sh, d_sh, sem_l0, sem_l1, sem_r0, sem_r1, sem_i0, sem_i1):
    cid = lax.axis_index("c")
    sid = lax.axis_index("s")
    pltpu.sync_copy(att_hbm.at[pl.ds(cid * FL, FL)], att_v)

    _zero_rows(xl0, FL)
    _zero_rows(dbuf, 16)
    base = sid * RPT
    _zero_shared(xl0, s_sh, base)
    _zero_shared(dbuf, d_sh, base)
    plsc.subcore_barrier()

    ebase = sid * EPT
    iot = lax.iota(_i32, 16)
    xl_t = xlh_hbm.at[cid]
    xr_t = xrh_hbm.at[cid]
    srcb = [src_idx0, src_idx1]
    dstb = [dst_idx0, dst_idx1]
    xlb = [xl0, xl1]
    xrb = [xr0, xr1]
    sl = [sem_l0, sem_l1]
    sr = [sem_r0, sem_r1]
    si = [sem_i0, sem_i1]
    att_hv = [att_v[pl.ds(h * C, C)] for h in range(HL)]
    lane_is = [iot == h for h in range(HL)]
    deg_col = jnp.where(iot == HL, jnp.ones((16,), _f32),
                        jnp.zeros((16,), _f32))

    def load_idx(b, p):
        off = ebase + b * K
        pltpu.sync_copy(src_hbm.at[pl.ds(off, K)], srcb[p])
        pltpu.sync_copy(dst_hbm.at[pl.ds(off, K)], dstb[p])

    def wait_idx(b, p):
        del b, p

    def fire(p):
        pltpu.async_copy(xl_t.at[srcb[p]], xlb[p], sl[p])
        pltpu.async_copy(xr_t.at[dstb[p]], xrb[p], sr[p])

    def drain(p):
        pltpu.make_async_copy(xl_t.at[srcb[p]], xlb[p], sl[p]).wait()
        pltpu.make_async_copy(xr_t.at[dstb[p]], xrb[p], sr[p]).wait()

    load_idx(0, 0)
    wait_idx(0, 0)
    fire(0)
    load_idx(1, 1)

    def pair(i, carry):
        for p in (0, 1):
            b = i * 2 + p
            q = 1 - p
            drain(p)
            wait_idx(b + 1, q)
            fire(q)
            xl_rows = xlb[p]
            xr_rows = xrb[p]

            @plsc.parallel_loop(0, K, unroll=4)
            def edge(e):
                wrow = deg_col
                for h in range(HL):
                    xlv = xl_rows[e, pl.ds(h * C, C)]
                    xrv = xr_rows[e, pl.ds(h * C, C)]
                    s = xlv + xrv
                    lk = jnp.maximum(s, s * _f32(0.2))
                    logit = jnp.sum(lk * att_hv[h])
                    wv = jnp.exp(jnp.broadcast_to(logit, (16,)))
                    msg_buf[e, pl.ds(h * C, C)] = xlv * wv
                    wrow = jnp.where(lane_is[h], wv, wrow)
                dbuf[e, :] = wrow
            pltpu.sync_copy(msg_buf, s_sh.at[dstb[p]], add=True)
            pltpu.sync_copy(dbuf, d_sh.at[dstb[p]], add=True)
            load_idx(b + 2, p)
        return carry

    lax.fori_loop(0, NB // 2, pair, 0)
    drain(0)
    wait_idx(NB, 0)
    wait_idx(NB + 1, 1)
    plsc.subcore_barrier()
    _copy_out(s_sh, s_out, cid, base)
    _copy_out(d_sh, d_out, cid, base)


_edge_call = functools.partial(
    pl.kernel,
    out_type=[jax.ShapeDtypeStruct((2, NP, FL), _f32),
              jax.ShapeDtypeStruct((2, NP, 16), _f32)],
    mesh=_MESH,
    compiler_params=_SC_PARAMS,
    scratch_types=[
        pltpu.VMEM((FL,), _f32),
        pltpu.VMEM((K,), _i32),
        pltpu.VMEM((K,), _i32),
        pltpu.VMEM((K,), _i32),
        pltpu.VMEM((K,), _i32),
        pltpu.VMEM((K, FL), _f32),
        pltpu.VMEM((K, FL), _f32),
        pltpu.VMEM((K, FL), _f32),
        pltpu.VMEM((K, FL), _f32),
        pltpu.VMEM((K, FL), _f32),
        pltpu.VMEM((K, 16), _f32),
        pltpu.VMEM_SHARED((NP, FL), _f32),
        pltpu.VMEM_SHARED((NP, 16), _f32),
        pltpu.SemaphoreType.DMA,
        pltpu.SemaphoreType.DMA,
        pltpu.SemaphoreType.DMA,
        pltpu.SemaphoreType.DMA,
        pltpu.SemaphoreType.DMA,
        pltpu.SemaphoreType.DMA,
    ],
)(_edge_pass)


def _gcn_pass(xgn_hbm, src_hbm, dst_hbm, o_out,
              src_idx0, src_idx1, dst_idx0, dst_idx1, rows0, rows1,
              o_sh, sem_a0, sem_a1, sem_i0, sem_i1):
    cid = lax.axis_index("c")
    sid = lax.axis_index("s")
    _zero_rows(rows0, FL)
    base = sid * RPT
    _zero_shared(rows0, o_sh, base)
    plsc.subcore_barrier()
    ebase = sid * EPT
    xg_t = xgn_hbm.at[cid]
    srcb = [src_idx0, src_idx1]
    dstb = [dst_idx0, dst_idx1]
    rowsb = [rows0, rows1]
    sems = [sem_a0, sem_a1]
    si = [sem_i0, sem_i1]

    def load_idx(b, p):
        off = ebase + b * K
        pltpu.sync_copy(src_hbm.at[pl.ds(off, K)], srcb[p])
        pltpu.sync_copy(dst_hbm.at[pl.ds(off, K)], dstb[p])

    def wait_idx(b, p):
        del b, p

    load_idx(0, 0)
    wait_idx(0, 0)
    pltpu.async_copy(xg_t.at[srcb[0]], rowsb[0], sems[0])
    load_idx(1, 1)

    def pair(i, carry):
        for p in (0, 1):
            b = i * 2 + p
            q = 1 - p
            pltpu.make_async_copy(xg_t.at[srcb[p]], rowsb[p], sems[p]).wait()
            wait_idx(b + 1, q)
            pltpu.async_copy(xg_t.at[srcb[q]], rowsb[q], sems[q])
            pltpu.sync_copy(rowsb[p], o_sh.at[dstb[p]], add=True)
            load_idx(b + 2, p)
        return carry

    lax.fori_loop(0, NB // 2, pair, 0)
    pltpu.make_async_copy(xg_t.at[srcb[0]], rowsb[0], sems[0]).wait()
    wait_idx(NB, 0)
    wait_idx(NB + 1, 1)
    plsc.subcore_barrier()
    _copy_out(o_sh, o_out, cid, base)


_gcn_call = functools.partial(
    pl.kernel,
    out_type=jax.ShapeDtypeStruct((2, NP, FL), _f32),
    mesh=_MESH,
    compiler_params=_SC_PARAMS,
    scratch_types=[
        pltpu.VMEM((K,), _i32),
        pltpu.VMEM((K,), _i32),
        pltpu.VMEM((K,), _i32),
        pltpu.VMEM((K,), _i32),
        pltpu.VMEM((K, FL), _f32),
        pltpu.VMEM((K, FL), _f32),
        pltpu.VMEM_SHARED((NP, FL), _f32),
        pltpu.SemaphoreType.DMA,
        pltpu.SemaphoreType.DMA,
        pltpu.SemaphoreType.DMA,
        pltpu.SemaphoreType.DMA,
    ],
)(_gcn_pass)


def _mm2_body(x_ref, wl_ref, wr_ref, xl_ref, xr_ref):
    x = x_ref[...]
    xl = jnp.dot(x, wl_ref[...], preferred_element_type=_f32)
    xr = jnp.dot(x, wr_ref[...], preferred_element_type=_f32)
    xl_ref[0] = xl[:, :FL]
    xl_ref[1] = xl[:, FL:]
    xr_ref[0] = xr[:, :FL]
    xr_ref[1] = xr[:, FL:]


def _mid_body(s_ref, d_ref, erep_ref, bias1_ref, gamma_ref,
              beta_ref, pw_ref, wg_ref, xgn_ref, dinv_ref):
    deg = d_ref[0, :N, HL:HL + 1]
    dinv = lax.rsqrt(jnp.maximum(deg, 1.0))
    pw = pw_ref[0, 0]
    erep = erep_ref[...]
    xg = None
    zpad = jnp.zeros((NP - N, FL), _f32)
    for i in range(2):
        denom = d_ref[i, :N, 0:HL]
        den64 = jnp.dot(denom, erep, preferred_element_type=_f32)
        h1 = s_ref[i, :N] / den64 + bias1_ref[i]
        mu = jnp.mean(h1, axis=0, keepdims=True)
        xc = h1 - mu
        var = jnp.mean(xc * xc, axis=0, keepdims=True)
        hbn = xc * lax.rsqrt(var + 1e-5) * gamma_ref[i] + beta_ref[i]
        hp = jnp.where(hbn >= 0, hbn, pw * hbn)
        part = jnp.dot(hp, wg_ref[pl.ds(i * FL, FL), :],
                       preferred_element_type=_f32)
        xg = part if xg is None else xg + part
    xgn = xg * dinv
    xgn_ref[0, :N] = xgn[:, :FL]
    xgn_ref[1, :N] = xgn[:, FL:]
    xgn_ref[0, N:] = zpad
    xgn_ref[1, N:] = zpad
    dinv_ref[...] = dinv


def _fin_body(o_ref, dinv_ref, bg_ref, out_ref):
    dinv = dinv_ref[...]
    bg = bg_ref[...]
    out_ref[:, :FL] = o_ref[0, :N] * dinv + bg[:, :FL]
    out_ref[:, FL:] = o_ref[1, :N] * dinv + bg[:, FL:]


def kernel(x, edge_index, Wl, Wr, att, bias1, gamma, beta, prelu_w, Wg, bias_g):
    xpad = jnp.pad(x, ((0, NP - N), (0, 0)))
    xlh, xrh = pl.pallas_call(
        _mm2_body,
        out_shape=[jax.ShapeDtypeStruct((2, NP, FL), _f32)] * 2,
    )(xpad, Wl, Wr)

    loops = jnp.arange(N, dtype=_i32)
    padi = jnp.full((EPAD - EFULL + 2 * K,), N, _i32)
    srcf = jnp.concatenate([edge_index[0], loops, padi])
    dstf = jnp.concatenate([edge_index[1], loops, padi])

    s_out, d_out = _edge_call(xlh, xrh, att.reshape(-1), srcf, dstf)

    erep = jnp.kron(jnp.eye(HL, dtype=_f32), jnp.ones((1, C), _f32))

    xgn_pad, dinv = pl.pallas_call(
        _mid_body,
        out_shape=[jax.ShapeDtypeStruct((2, NP, FL), _f32),
                   jax.ShapeDtypeStruct((N, 1), _f32)],
    )(s_out, d_out, erep, bias1.reshape(2, 1, FL), gamma.reshape(2, 1, FL),
      beta.reshape(2, 1, FL), prelu_w.reshape(1, 1), Wg)

    o_out = _gcn_call(xgn_pad, srcf, dstf)

    out = pl.pallas_call(
        _fin_body,
        out_shape=jax.ShapeDtypeStruct((N, OUT), _f32),
    )(o_out, dinv, bias_g.reshape(1, -1))
    return out
